# Initial kernel scaffold; baseline (speedup 1.0000x reference)
import jax
import jax.numpy as jnp
from jax.experimental import pallas as pl


def kernel(samples, targets, train_features, train_labels):
    raise NotImplementedError("write your pallas kernel here")


# speedup vs baseline: 1.1420x; 1.1420x over previous
import functools

import jax
import jax.numpy as jnp
from jax import lax
from jax.experimental import pallas as pl
from jax.experimental.pallas import tpu as pltpu

K_NB = 16
N_CLASSES = 1000
QB = 256
CHUNK = 2048

NEG = -1e30


def _extract_topk(x, xl, width, out_v_ref, out_l_ref):
    rows = x.shape[0]
    lane = lax.broadcasted_iota(jnp.int32, (rows, width), 1)
    for j in range(K_NB):
        m = jnp.max(x, axis=1, keepdims=True)
        ism = x >= m
        first = jnp.min(jnp.where(ism, lane, width), axis=1, keepdims=True)
        sel = lane == first
        labj = jnp.max(jnp.where(sel, xl, -1), axis=1, keepdims=True)
        out_v_ref[:, j : j + 1] = m
        out_l_ref[:, j : j + 1] = labj
        x = jnp.where(sel, NEG, x)


def _topk_body(n_real, nchunk, s_ref, tf_ref, lab_ref, ov_ref, ol_ref,
               rv_ref, rl_ref, cv_ref, cl_ref):
    c = pl.program_id(1)

    @pl.when(c == 0)
    def _init():
        rv_ref[...] = jnp.full((QB, K_NB), NEG, jnp.float32)
        rl_ref[...] = jnp.zeros((QB, K_NB), jnp.int32)

    sims = lax.dot_general(
        s_ref[...], tf_ref[...], (((1,), (1,)), ((), ())),
        preferred_element_type=jnp.float32)
    col = c * CHUNK + lax.broadcasted_iota(jnp.int32, (QB, CHUNK), 1)
    sims = jnp.where(col < n_real, sims, NEG)
    labs = jnp.broadcast_to(lab_ref[0], (QB, CHUNK))

    _extract_topk(sims, labs, CHUNK, cv_ref, cl_ref)

    y = jnp.concatenate([rv_ref[...], cv_ref[...]], axis=1)
    yl = jnp.concatenate([rl_ref[...], cl_ref[...]], axis=1)
    _extract_topk(y, yl, 2 * K_NB, rv_ref, rl_ref)

    @pl.when(c == nchunk - 1)
    def _out():
        ov_ref[...] = rv_ref[...]
        ol_ref[...] = rl_ref[...]


def _vote_body(v_ref, l_ref, o_ref):
    vals = v_ref[...]
    labs = l_ref[...]
    cls = lax.broadcasted_iota(jnp.int32, (QB, 1024), 1)
    acc = jnp.zeros((QB, 1024), jnp.float32)
    for j in range(K_NB):
        acc = acc + jnp.where(labs[:, j : j + 1] == cls,
                              vals[:, j : j + 1], 0.0)
    rs = jnp.sum(acc, axis=1, keepdims=True)
    rs = jnp.where(rs == 0.0, 1.0, rs)
    o_ref[...] = acc / rs


def kernel(samples, targets, train_features, train_labels):
    q, d = samples.shape
    n = train_features.shape[0]
    nchunk = (n + CHUNK - 1) // CHUNK
    npad = nchunk * CHUNK
    nqb = q // QB

    tf = jnp.pad(train_features, ((0, npad - n), (0, 0)))
    labs = jnp.pad(train_labels.astype(jnp.int32), (0, npad - n))
    labs3 = labs.reshape(nchunk, 1, CHUNK)

    topk_v, topk_l = pl.pallas_call(
        functools.partial(_topk_body, n, nchunk),
        grid=(nqb, nchunk),
        in_specs=[
            pl.BlockSpec((QB, d), lambda b, c: (b, 0)),
            pl.BlockSpec((CHUNK, d), lambda b, c: (c, 0)),
            pl.BlockSpec((1, 1, CHUNK), lambda b, c: (c, 0, 0)),
        ],
        out_specs=[
            pl.BlockSpec((QB, K_NB), lambda b, c: (b, 0)),
            pl.BlockSpec((QB, K_NB), lambda b, c: (b, 0)),
        ],
        out_shape=[
            jax.ShapeDtypeStruct((q, K_NB), jnp.float32),
            jax.ShapeDtypeStruct((q, K_NB), jnp.int32),
        ],
        scratch_shapes=[
            pltpu.VMEM((QB, K_NB), jnp.float32),
            pltpu.VMEM((QB, K_NB), jnp.int32),
            pltpu.VMEM((QB, K_NB), jnp.float32),
            pltpu.VMEM((QB, K_NB), jnp.int32),
        ],
    )(samples, tf, labs3)

    probas = pl.pallas_call(
        _vote_body,
        grid=(nqb,),
        in_specs=[
            pl.BlockSpec((QB, K_NB), lambda b: (b, 0)),
            pl.BlockSpec((QB, K_NB), lambda b: (b, 0)),
        ],
        out_specs=pl.BlockSpec((QB, 1024), lambda b: (b, 0)),
        out_shape=jax.ShapeDtypeStruct((q, 1024), jnp.float32),
    )(topk_v, topk_l)

    return probas[:, :N_CLASSES], targets


# speedup vs baseline: 4.4124x; 3.8638x over previous
import functools

import jax
import jax.numpy as jnp
from jax import lax
from jax.experimental import pallas as pl
from jax.experimental.pallas import tpu as pltpu
from jax.experimental.pallas import tpu_sc as plsc

K_NB = 16
N_CLASSES = 1000
QB = 256
CHUNK = 2048
NEG = -1e30

NC, NS, L = 2, 16, 16
NW = NC * NS

NPAD = 102400
NG = NPAD // L
NG_CAP = 48
CAND = NG_CAP * L


def _matmul_body(n_real, s_ref, tf_ref, o_ref, m_ref):
    c = pl.program_id(1)
    sims = lax.dot_general(
        s_ref[...], tf_ref[...], (((1,), (1,)), ((), ())),
        preferred_element_type=jnp.float32)
    col = c * CHUNK + lax.broadcasted_iota(jnp.int32, (QB, CHUNK), 1)
    sims = jnp.where(col < n_real, sims, NEG)
    o_ref[...] = sims
    m_ref[...] = jnp.max(sims.reshape(QB, L, CHUNK // L), axis=1)


def _thresh_body(m_ref, t_ref):
    x = m_ref[...]
    lane = lax.broadcasted_iota(jnp.int32, (QB, NG), 1)
    for j in range(K_NB):
        m = jnp.max(x, axis=1, keepdims=True)
        if j < K_NB - 1:
            ism = x >= m
            first = jnp.min(jnp.where(ism, lane, NG), axis=1, keepdims=True)
            x = jnp.where(lane == first, NEG, x)
    t_ref[...] = jnp.broadcast_to(m, (QB, L))


def _scalar_max16(v):
    m = jnp.maximum(v, lax.rev(v, (0,)))
    s = m[0]
    for i in range(1, 8):
        s = jnp.maximum(s, m[i])
    return s


def _sc_cand_body(rows_per_tile, sims_ref, labels_ref, m_ref, th_ref,
                  cv_ref, cl_ref, cn_ref,
                  mb0, mb1, cv0, cv1, cl0, cl1, cntv, thall,
                  glist, msem0, msem1, gv0, gv1, gl0, gl1):
    wid = lax.axis_index("s") * NC + lax.axis_index("c")
    row0 = wid * rows_per_tile
    iota = lax.iota(jnp.int32, L)

    pltpu.sync_copy(th_ref.at[pl.ds(row0 * L, rows_per_tile * L)], thall)

    def fire_m(r, dst, sem):
        pltpu.make_async_copy(
            m_ref.at[pl.ds((row0 + r) * NG, NG)], dst, sem).start()

    fire_m(0, mb0, msem0)

    def do_row(r, mb, other_mb, cv, cl, gv, gl, msem, other_msem):
        @pl.when(r + 1 < rows_per_tile)
        def _():
            fire_m(r + 1, other_mb, other_msem)

        pltpu.make_async_copy(
            m_ref.at[pl.ds(0, NG)], mb, msem).wait()

        @pl.when(r >= 2)
        def _():
            pltpu.make_async_copy(
                sims_ref.at[pl.ds(0, CAND)], cv, gv).wait()
            pltpu.make_async_copy(
                labels_ref.at[pl.ds(0, CAND)], cl, gl).wait()
            out = (row0 + r - 2) * CAND
            pltpu.sync_copy(cv, cv_ref.at[pl.ds(out, CAND)])
            pltpu.sync_copy(cl, cl_ref.at[pl.ds(out, CAND)])

        tv = thall[pl.ds(r * L, L)]
        ts = tv[0]
        glist[NG_CAP] = 0

        def sb_body(sb, _):
            off = sb * 128
            vs = [mb[pl.ds(off + k * L, L)] for k in range(8)]
            mm = vs[0]
            for k in range(1, 8):
                mm = jnp.maximum(mm, vs[k])

            @pl.when(_scalar_max16(mm) >= ts)
            def _():
                for k in range(8):
                    vk = vs[k]

                    @pl.when(_scalar_max16(vk) >= ts)
                    def _(vk=vk, k=k):
                        for i in range(L):
                            @pl.when(vk[i] >= ts)
                            def _(i=i, k=k):
                                cnt = glist[NG_CAP]

                                @pl.when(cnt < NG_CAP)
                                def _():
                                    glist[cnt] = off + k * L + i
                                glist[NG_CAP] = cnt + 1
            return 0

        lax.fori_loop(0, NG // 128, sb_body, 0)

        cnt = jnp.minimum(glist[NG_CAP], NG_CAP)
        cntv[pl.ds(r * L, L)] = jnp.full((L,), cnt, jnp.int32)

        def fire_g(j, _):
            g = glist[jnp.minimum(j, cnt - 1)]
            cc = lax.shift_right_logical(g, 7)
            ll = lax.bitwise_and(g, 127)
            base = cc * CHUNK + ll
            lidx = jnp.full((L,), base, jnp.int32) + 128 * iota
            sidx = lidx + jnp.full((L,), (row0 + r) * NPAD, jnp.int32)
            pltpu.async_copy(sims_ref.at[sidx],
                             cv.at[pl.ds(j * L, L)], gv)
            pltpu.async_copy(labels_ref.at[lidx],
                             cl.at[pl.ds(j * L, L)], gl)
            return 0

        lax.fori_loop(0, NG_CAP, fire_g, 0)

    def pair_body(u, _):
        do_row(2 * u, mb0, mb1, cv0, cl0, gv0, gl0, msem0, msem1)
        do_row(2 * u + 1, mb1, mb0, cv1, cl1, gv1, gl1, msem1, msem0)
        return 0

    lax.fori_loop(0, rows_per_tile // 2, pair_body, 0)

    for p, (cv, cl, gv, gl) in enumerate(((cv0, cl0, gv0, gl0),
                                          (cv1, cl1, gv1, gl1))):
        pltpu.make_async_copy(sims_ref.at[pl.ds(0, CAND)], cv, gv).wait()
        pltpu.make_async_copy(labels_ref.at[pl.ds(0, CAND)], cl, gl).wait()
        out = (row0 + rows_per_tile - 2 + p) * CAND
        pltpu.sync_copy(cv, cv_ref.at[pl.ds(out, CAND)])
        pltpu.sync_copy(cl, cl_ref.at[pl.ds(out, CAND)])
    pltpu.sync_copy(cntv, cn_ref.at[pl.ds(row0 * L, rows_per_tile * L)])


def _final_body(v_ref, l_ref, n_ref, o_ref):
    lane = lax.broadcasted_iota(jnp.int32, (QB, CAND), 1)
    cnt = n_ref[...][:, 0:1]
    x = jnp.where(lane < cnt * L, v_ref[...], NEG)
    xl = l_ref[...]
    cls = lax.broadcasted_iota(jnp.int32, (QB, 1024), 1)
    acc = jnp.zeros((QB, 1024), jnp.float32)
    for j in range(K_NB):
        m = jnp.max(x, axis=1, keepdims=True)
        ism = x >= m
        first = jnp.min(jnp.where(ism, lane, CAND), axis=1, keepdims=True)
        sel = lane == first
        labj = jnp.max(jnp.where(sel, xl, -1), axis=1, keepdims=True)
        acc = acc + jnp.where(labj == cls, m, 0.0)
        x = jnp.where(sel, NEG, x)
    rs = jnp.sum(acc, axis=1, keepdims=True)
    rs = jnp.where(rs == 0.0, 1.0, rs)
    o_ref[...] = acc / rs


def kernel(samples, targets, train_features, train_labels):
    q, d = samples.shape
    n = train_features.shape[0]
    nchunk = NPAD // CHUNK
    nqb = q // QB
    rows_per_tile = q // NW

    tf = jnp.pad(train_features, ((0, NPAD - n), (0, 0)))
    labs_i32 = jnp.pad(train_labels.astype(jnp.int32), (0, NPAD - n))

    sims, gmax = pl.pallas_call(
        functools.partial(_matmul_body, n),
        grid=(nqb, nchunk),
        in_specs=[
            pl.BlockSpec((QB, d), lambda b, c: (b, 0)),
            pl.BlockSpec((CHUNK, d), lambda b, c: (c, 0)),
        ],
        out_specs=[
            pl.BlockSpec((QB, CHUNK), lambda b, c: (b, c)),
            pl.BlockSpec((QB, CHUNK // L), lambda b, c: (b, c)),
        ],
        out_shape=[
            jax.ShapeDtypeStruct((q, NPAD), jnp.float32),
            jax.ShapeDtypeStruct((q, NG), jnp.float32),
        ],
    )(samples, tf)

    thresh = pl.pallas_call(
        _thresh_body,
        grid=(nqb,),
        in_specs=[pl.BlockSpec((QB, NG), lambda b: (b, 0))],
        out_specs=pl.BlockSpec((QB, L), lambda b: (b, 0)),
        out_shape=jax.ShapeDtypeStruct((q, L), jnp.float32),
    )(gmax)

    mesh = plsc.VectorSubcoreMesh(core_axis_name="c", subcore_axis_name="s")
    candv, candl, cnts = pl.kernel(
        functools.partial(_sc_cand_body, rows_per_tile),
        out_type=[
            jax.ShapeDtypeStruct((q * CAND,), jnp.float32),
            jax.ShapeDtypeStruct((q * CAND,), jnp.int32),
            jax.ShapeDtypeStruct((q * L,), jnp.int32),
        ],
        mesh=mesh,
        scratch_types=[
            pltpu.VMEM((NG,), jnp.float32),
            pltpu.VMEM((NG,), jnp.float32),
            pltpu.VMEM((CAND,), jnp.float32),
            pltpu.VMEM((CAND,), jnp.float32),
            pltpu.VMEM((CAND,), jnp.int32),
            pltpu.VMEM((CAND,), jnp.int32),
            pltpu.VMEM((rows_per_tile * L,), jnp.int32),
            pltpu.VMEM((rows_per_tile * L,), jnp.float32),
            pltpu.SMEM((NG_CAP + 8,), jnp.int32),
            pltpu.SemaphoreType.DMA,
            pltpu.SemaphoreType.DMA,
            pltpu.SemaphoreType.DMA,
            pltpu.SemaphoreType.DMA,
            pltpu.SemaphoreType.DMA,
            pltpu.SemaphoreType.DMA,
        ],
    )(sims.reshape(q * NPAD), labs_i32, gmax.reshape(q * NG),
      thresh.reshape(q * L))

    probas = pl.pallas_call(
        _final_body,
        grid=(nqb,),
        in_specs=[
            pl.BlockSpec((QB, CAND), lambda b: (b, 0)),
            pl.BlockSpec((QB, CAND), lambda b: (b, 0)),
            pl.BlockSpec((QB, L), lambda b: (b, 0)),
        ],
        out_specs=pl.BlockSpec((QB, 1024), lambda b: (b, 0)),
        out_shape=jax.ShapeDtypeStruct((q, 1024), jnp.float32),
    )(candv.reshape(q, CAND), candl.reshape(q, CAND), cnts.reshape(q, L))

    return probas[:, :N_CLASSES], targets


# speedup vs baseline: 5.3452x; 1.2114x over previous
import functools

import jax
import jax.numpy as jnp
from jax import lax
from jax.experimental import pallas as pl
from jax.experimental.pallas import tpu as pltpu
from jax.experimental.pallas import tpu_sc as plsc

K_NB = 16
N_CLASSES = 1000
QB = 256
CHUNK = 2048
NEG = -1e30

NC, NS, L = 2, 16, 16
NW = NC * NS

NPAD = 102400
NCHUNK = NPAD // CHUNK
NG = NPAD // L
NG_CAP = 48
CAND = NG_CAP * L


def _matmul_body(n_real, s_ref, tf_ref, o_ref, m_ref):
    c = pl.program_id(1)
    sims = lax.dot_general(
        s_ref[...], tf_ref[...], (((1,), (1,)), ((), ())),
        preferred_element_type=jnp.float32)
    col = c * CHUNK + lax.broadcasted_iota(jnp.int32, (QB, CHUNK), 1)
    sims = jnp.where(col < n_real, sims, NEG)
    o_ref[...] = sims.reshape(QB * CHUNK)
    m_ref[...] = jnp.max(sims.reshape(QB, L, CHUNK // L), axis=1)


def _thresh_body(m_ref, t_ref):
    x = m_ref[...]
    lane = lax.broadcasted_iota(jnp.int32, (QB, NG), 1)
    for j in range(K_NB):
        m = jnp.max(x, axis=1, keepdims=True)
        if j < K_NB - 1:
            ism = x >= m
            first = jnp.min(jnp.where(ism, lane, NG), axis=1, keepdims=True)
            x = jnp.where(lane == first, NEG, x)
    t_ref[...] = jnp.broadcast_to(m, (QB, L))


def _scalar_max16(v):
    m = jnp.maximum(v, lax.rev(v, (0,)))
    s = m[0]
    for i in range(1, 8):
        s = jnp.maximum(s, m[i])
    return s


def _sc_cand_body(rows_per_tile, sims_ref, labels_ref, m_ref, th_ref,
                  cv_ref, cl_ref, cn_ref,
                  mb0, mb1, cv0, cv1, cl0, cl1, cntv, thall,
                  glist, msem0, msem1, gv0, gv1, gl0, gl1):
    wid = lax.axis_index("s") * NC + lax.axis_index("c")
    row0 = wid * rows_per_tile
    iota = lax.iota(jnp.int32, L)

    pltpu.sync_copy(th_ref.at[pl.ds(row0 * L, rows_per_tile * L)], thall)

    def fire_m(r, dst, sem):
        pltpu.make_async_copy(
            m_ref.at[pl.ds((row0 + r) * NG, NG)], dst, sem).start()

    fire_m(0, mb0, msem0)

    def do_row(r, mb, other_mb, cv, cl, gv, gl, msem, other_msem):
        @pl.when(r + 1 < rows_per_tile)
        def _():
            fire_m(r + 1, other_mb, other_msem)

        pltpu.make_async_copy(
            m_ref.at[pl.ds(0, NG)], mb, msem).wait()

        @pl.when(r >= 2)
        def _():
            pltpu.make_async_copy(
                sims_ref.at[pl.ds(0, CAND)], cv, gv).wait()
            pltpu.make_async_copy(
                labels_ref.at[pl.ds(0, CAND)], cl, gl).wait()
            out = (row0 + r - 2) * CAND
            pltpu.sync_copy(cv, cv_ref.at[pl.ds(out, CAND)])
            pltpu.sync_copy(cl, cl_ref.at[pl.ds(out, CAND)])

        tv = thall[pl.ds(r * L, L)]
        ts = tv[0]
        glist[NG_CAP] = 0

        def sb_body(sb, _):
            off = sb * 128
            vs = [mb[pl.ds(off + k * L, L)] for k in range(8)]
            mm = vs[0]
            for k in range(1, 8):
                mm = jnp.maximum(mm, vs[k])

            @pl.when(_scalar_max16(mm) >= ts)
            def _():
                for k in range(8):
                    vk = vs[k]

                    @pl.when(_scalar_max16(vk) >= ts)
                    def _(vk=vk, k=k):
                        for i in range(L):
                            @pl.when(vk[i] >= ts)
                            def _(i=i, k=k):
                                cnt = glist[NG_CAP]

                                @pl.when(cnt < NG_CAP)
                                def _():
                                    glist[cnt] = off + k * L + i
                                glist[NG_CAP] = cnt + 1
            return 0

        lax.fori_loop(0, NG // 128, sb_body, 0)

        cnt = jnp.minimum(glist[NG_CAP], NG_CAP)
        cntv[pl.ds(r * L, L)] = jnp.full((L,), cnt, jnp.int32)

        rg = row0 + r
        sbase0 = ((lax.shift_right_logical(rg, 8) * NCHUNK) * QB
                  + lax.bitwise_and(rg, QB - 1)) * CHUNK

        def fire_g(j, _):
            g = glist[jnp.minimum(j, cnt - 1)]
            cc = lax.shift_right_logical(g, 7)
            ll = lax.bitwise_and(g, 127)
            base = cc * CHUNK + ll
            lidx = jnp.full((L,), base, jnp.int32) + 128 * iota
            sidx = jnp.full((L,), sbase0 + cc * (QB * CHUNK) + ll,
                            jnp.int32) + 128 * iota
            pltpu.async_copy(sims_ref.at[sidx],
                             cv.at[pl.ds(j * L, L)], gv)
            pltpu.async_copy(labels_ref.at[lidx],
                             cl.at[pl.ds(j * L, L)], gl)
            return 0

        lax.fori_loop(0, NG_CAP, fire_g, 0)

    def pair_body(u, _):
        do_row(2 * u, mb0, mb1, cv0, cl0, gv0, gl0, msem0, msem1)
        do_row(2 * u + 1, mb1, mb0, cv1, cl1, gv1, gl1, msem1, msem0)
        return 0

    lax.fori_loop(0, rows_per_tile // 2, pair_body, 0)

    for p, (cv, cl, gv, gl) in enumerate(((cv0, cl0, gv0, gl0),
                                          (cv1, cl1, gv1, gl1))):
        pltpu.make_async_copy(sims_ref.at[pl.ds(0, CAND)], cv, gv).wait()
        pltpu.make_async_copy(labels_ref.at[pl.ds(0, CAND)], cl, gl).wait()
        out = (row0 + rows_per_tile - 2 + p) * CAND
        pltpu.sync_copy(cv, cv_ref.at[pl.ds(out, CAND)])
        pltpu.sync_copy(cl, cl_ref.at[pl.ds(out, CAND)])
    pltpu.sync_copy(cntv, cn_ref.at[pl.ds(row0 * L, rows_per_tile * L)])


def _final_body(v_ref, l_ref, n_ref, o_ref):
    lane = lax.broadcasted_iota(jnp.int32, (QB, CAND), 1)
    cnt = n_ref[...][:, 0:1]
    x = jnp.where(lane < cnt * L, v_ref[...], NEG)
    xl = l_ref[...]
    cls = lax.broadcasted_iota(jnp.int32, (QB, 1024), 1)
    acc = jnp.zeros((QB, 1024), jnp.float32)
    for j in range(K_NB):
        m = jnp.max(x, axis=1, keepdims=True)
        ism = x >= m
        first = jnp.min(jnp.where(ism, lane, CAND), axis=1, keepdims=True)
        sel = lane == first
        labj = jnp.max(jnp.where(sel, xl, -1), axis=1, keepdims=True)
        acc = acc + jnp.where(labj == cls, m, 0.0)
        x = jnp.where(sel, NEG, x)
    rs = jnp.sum(acc, axis=1, keepdims=True)
    rs = jnp.where(rs == 0.0, 1.0, rs)
    o_ref[...] = acc / rs


def kernel(samples, targets, train_features, train_labels):
    q, d = samples.shape
    n = train_features.shape[0]
    nchunk = NPAD // CHUNK
    nqb = q // QB
    rows_per_tile = q // NW

    tf = jnp.pad(train_features, ((0, NPAD - n), (0, 0)))
    labs_i32 = jnp.pad(train_labels.astype(jnp.int32), (0, NPAD - n))

    sims, gmax = pl.pallas_call(
        functools.partial(_matmul_body, n),
        grid=(nqb, nchunk),
        in_specs=[
            pl.BlockSpec((QB, d), lambda b, c: (b, 0)),
            pl.BlockSpec((CHUNK, d), lambda b, c: (c, 0)),
        ],
        out_specs=[
            pl.BlockSpec((QB * CHUNK,), lambda b, c: (b * NCHUNK + c,)),
            pl.BlockSpec((QB, CHUNK // L), lambda b, c: (b, c)),
        ],
        out_shape=[
            jax.ShapeDtypeStruct((q * NPAD,), jnp.float32),
            jax.ShapeDtypeStruct((q, NG), jnp.float32),
        ],
    )(samples, tf)

    thresh = pl.pallas_call(
        _thresh_body,
        grid=(nqb,),
        in_specs=[pl.BlockSpec((QB, NG), lambda b: (b, 0))],
        out_specs=pl.BlockSpec((QB, L), lambda b: (b, 0)),
        out_shape=jax.ShapeDtypeStruct((q, L), jnp.float32),
    )(gmax)

    mesh = plsc.VectorSubcoreMesh(core_axis_name="c", subcore_axis_name="s")
    candv, candl, cnts = pl.kernel(
        functools.partial(_sc_cand_body, rows_per_tile),
        out_type=[
            jax.ShapeDtypeStruct((q * CAND,), jnp.float32),
            jax.ShapeDtypeStruct((q * CAND,), jnp.int32),
            jax.ShapeDtypeStruct((q * L,), jnp.int32),
        ],
        mesh=mesh,
        scratch_types=[
            pltpu.VMEM((NG,), jnp.float32),
            pltpu.VMEM((NG,), jnp.float32),
            pltpu.VMEM((CAND,), jnp.float32),
            pltpu.VMEM((CAND,), jnp.float32),
            pltpu.VMEM((CAND,), jnp.int32),
            pltpu.VMEM((CAND,), jnp.int32),
            pltpu.VMEM((rows_per_tile * L,), jnp.int32),
            pltpu.VMEM((rows_per_tile * L,), jnp.float32),
            pltpu.SMEM((NG_CAP + 8,), jnp.int32),
            pltpu.SemaphoreType.DMA,
            pltpu.SemaphoreType.DMA,
            pltpu.SemaphoreType.DMA,
            pltpu.SemaphoreType.DMA,
            pltpu.SemaphoreType.DMA,
            pltpu.SemaphoreType.DMA,
        ],
    )(sims, labs_i32, gmax.reshape(q * NG), thresh.reshape(q * L))

    probas = pl.pallas_call(
        _final_body,
        grid=(nqb,),
        in_specs=[
            pl.BlockSpec((QB, CAND), lambda b: (b, 0)),
            pl.BlockSpec((QB, CAND), lambda b: (b, 0)),
            pl.BlockSpec((QB, L), lambda b: (b, 0)),
        ],
        out_specs=pl.BlockSpec((QB, 1024), lambda b: (b, 0)),
        out_shape=jax.ShapeDtypeStruct((q, 1024), jnp.float32),
    )(candv.reshape(q, CAND), candl.reshape(q, CAND), cnts.reshape(q, L))

    return probas[:, :N_CLASSES], targets
